# Initial kernel scaffold; baseline (speedup 1.0000x reference)
import jax
import jax.numpy as jnp
from jax.experimental import pallas as pl


def kernel(embedding_matrix, title_ids, desc_ids):
    raise NotImplementedError("write your pallas kernel here")


# speedup vs baseline: 3.9994x; 3.9994x over previous
import functools

import jax
import jax.numpy as jnp
from jax import lax
from jax.experimental import pallas as pl
from jax.experimental.pallas import tpu as pltpu
from jax.experimental.pallas import tpu_sc as plsc

VOCAB = 100000
D = 64
B = 4096
L = 50
TOTAL = 2 * B * L
NC = 2
NS = 16
NW = NC * NS
ROWS_PER_W = TOTAL // NW
BLK = 128
NBLK = 4
CHUNK = BLK * NBLK
NCHUNKS = ROWS_PER_W // CHUNK

_mesh = plsc.VectorSubcoreMesh(core_axis_name="c", subcore_axis_name="s")


@functools.partial(
    pl.kernel,
    mesh=_mesh,
    out_type=jax.ShapeDtypeStruct((TOTAL, D), jnp.float32),
    compiler_params=pltpu.CompilerParams(use_tc_tiling_on_sc=False),
    scratch_types=[
        pltpu.VMEM((CHUNK,), jnp.int32),
        pltpu.VMEM((CHUNK, D), jnp.float32),
        pltpu.SemaphoreType.DMA,
    ],
)
def _gather_rows(idx_hbm, table_hbm, out_hbm, idx_v, rows_v, sem):
    wid = lax.axis_index("s") * NC + lax.axis_index("c")
    base = wid * ROWS_PER_W

    def chunk_body(c, _):
        off = pl.multiple_of(base + c * CHUNK, CHUNK)
        pltpu.sync_copy(idx_hbm.at[pl.ds(off, CHUNK)], idx_v)
        copies = []
        for j in range(NBLK):
            copies.append(
                pltpu.async_copy(
                    table_hbm.at[idx_v.at[pl.ds(j * BLK, BLK)]],
                    rows_v.at[pl.ds(j * BLK, BLK)],
                    sem,
                )
            )
        for cp in copies:
            cp.wait()
        pltpu.sync_copy(rows_v, out_hbm.at[pl.ds(off, CHUNK)])
        return ()

    lax.fori_loop(0, NCHUNKS, chunk_body, ())


def kernel(embedding_matrix, title_ids, desc_ids):
    t_idx = jnp.transpose(title_ids).reshape(-1).astype(jnp.int32)
    d_idx = jnp.transpose(desc_ids).reshape(-1).astype(jnp.int32)
    flat_idx = jnp.concatenate([t_idx, d_idx])
    out = _gather_rows(flat_idx, embedding_matrix)
    title_emb = out[: B * L].reshape(L, B, D)
    desc_emb = out[B * L :].reshape(L, B, D)
    return (title_emb, desc_emb)


# speedup vs baseline: 4.1906x; 1.0478x over previous
import functools

import jax
import jax.numpy as jnp
from jax import lax
from jax.experimental import pallas as pl
from jax.experimental.pallas import tpu as pltpu
from jax.experimental.pallas import tpu_sc as plsc

VOCAB = 100000
D = 64
B = 4096
L = 50
TOTAL = 2 * B * L
NC = 2
NS = 16
NW = NC * NS
ROWS_PER_W = TOTAL // NW
BLK = 128
NBLK = 5
CHUNK = BLK * NBLK
NCHUNKS = ROWS_PER_W // CHUNK
NBUF = 2

_mesh = plsc.VectorSubcoreMesh(core_axis_name="c", subcore_axis_name="s")


@functools.partial(
    pl.kernel,
    mesh=_mesh,
    out_type=jax.ShapeDtypeStruct((TOTAL, D), jnp.float32),
    compiler_params=pltpu.CompilerParams(use_tc_tiling_on_sc=False),
    scratch_types=[
        pltpu.VMEM((CHUNK,), jnp.int32),
        pltpu.VMEM((CHUNK,), jnp.int32),
        pltpu.VMEM((CHUNK, D), jnp.float32),
        pltpu.VMEM((CHUNK, D), jnp.float32),
        pltpu.SemaphoreType.DMA,
        pltpu.SemaphoreType.DMA,
        pltpu.SemaphoreType.DMA,
        pltpu.SemaphoreType.DMA,
        pltpu.SemaphoreType.DMA,
        pltpu.SemaphoreType.DMA,
    ],
)
def _gather_rows(idx_hbm, table_hbm, out_hbm,
                 idx_v0, idx_v1, rows_v0, rows_v1,
                 sem_i0, sem_i1, sem_g0, sem_g1, sem_o0, sem_o1):
    idx_v = (idx_v0, idx_v1)
    rows_v = (rows_v0, rows_v1)
    sem_i = (sem_i0, sem_i1)
    sem_g = (sem_g0, sem_g1)
    sem_o = (sem_o0, sem_o1)

    wid = lax.axis_index("s") * NC + lax.axis_index("c")
    base = wid * ROWS_PER_W

    def start_idx(c, b):
        off = pl.multiple_of(base + c * CHUNK, CHUNK)
        pltpu.make_async_copy(
            idx_hbm.at[pl.ds(off, CHUNK)], idx_v[b], sem_i[b]
        ).start()

    def wait_idx(b):
        pltpu.make_async_copy(
            idx_hbm.at[pl.ds(0, CHUNK)], idx_v[b], sem_i[b]
        ).wait()

    def start_gathers(b):
        for j in range(NBLK):
            pltpu.make_async_copy(
                table_hbm.at[idx_v[b].at[pl.ds(j * BLK, BLK)]],
                rows_v[b].at[pl.ds(j * BLK, BLK)],
                sem_g[b],
            ).start()

    def wait_gathers(b):
        for j in range(NBLK):
            pltpu.make_async_copy(
                table_hbm.at[idx_v[b].at[pl.ds(j * BLK, BLK)]],
                rows_v[b].at[pl.ds(j * BLK, BLK)],
                sem_g[b],
            ).wait()

    def start_out(c, b):
        off = pl.multiple_of(base + c * CHUNK, CHUNK)
        pltpu.make_async_copy(
            rows_v[b], out_hbm.at[pl.ds(off, CHUNK)], sem_o[b]
        ).start()

    def wait_out(b):
        pltpu.make_async_copy(
            rows_v[b], out_hbm.at[pl.ds(0, CHUNK)], sem_o[b]
        ).wait()

    for b in range(NBUF):
        start_idx(b, b)

    def outer(i, _):
        c0 = i * NBUF
        for b in range(NBUF):
            c = c0 + b
            @pl.when(c >= NBUF)
            def _():
                wait_out(b)

            wait_idx(b)
            start_gathers(b)
            wait_gathers(b)
            start_out(c, b)

            @pl.when(c + NBUF < NCHUNKS)
            def _():
                start_idx(c + NBUF, b)

        return ()

    lax.fori_loop(0, NCHUNKS // NBUF, outer, ())

    for b in range(NBUF):
        wait_out(b)


def kernel(embedding_matrix, title_ids, desc_ids):
    t_idx = jnp.transpose(title_ids).reshape(-1).astype(jnp.int32)
    d_idx = jnp.transpose(desc_ids).reshape(-1).astype(jnp.int32)
    flat_idx = jnp.concatenate([t_idx, d_idx])
    out = _gather_rows(flat_idx, embedding_matrix)
    title_emb = out[: B * L].reshape(L, B, D)
    desc_emb = out[B * L :].reshape(L, B, D)
    return (title_emb, desc_emb)


# speedup vs baseline: 5.8685x; 1.4004x over previous
import functools

import jax
import jax.numpy as jnp
from jax import lax
from jax.experimental import pallas as pl
from jax.experimental.pallas import tpu as pltpu
from jax.experimental.pallas import tpu_sc as plsc

VOCAB = 100000
D = 64
B = 4096
L = 50
ROWS = B * L
NC = 2
NS = 16
NW = NC * NS
WPA = NW // 2
ROWS_PER_W = ROWS // WPA
BLK = 128
NBLK = 5
CHUNK = BLK * NBLK
NCHUNKS = ROWS_PER_W // CHUNK
NBUF = 2

_mesh = plsc.VectorSubcoreMesh(core_axis_name="c", subcore_axis_name="s")


@functools.partial(
    pl.kernel,
    mesh=_mesh,
    out_type=(
        jax.ShapeDtypeStruct((ROWS, D), jnp.float32),
        jax.ShapeDtypeStruct((ROWS, D), jnp.float32),
    ),
    compiler_params=pltpu.CompilerParams(use_tc_tiling_on_sc=False),
    scratch_types=[
        pltpu.VMEM((CHUNK,), jnp.int32),
        pltpu.VMEM((CHUNK,), jnp.int32),
        pltpu.VMEM((CHUNK, D), jnp.float32),
        pltpu.VMEM((CHUNK, D), jnp.float32),
        pltpu.SemaphoreType.DMA,
        pltpu.SemaphoreType.DMA,
        pltpu.SemaphoreType.DMA,
        pltpu.SemaphoreType.DMA,
        pltpu.SemaphoreType.DMA,
        pltpu.SemaphoreType.DMA,
    ],
)
def _gather_rows(t_idx_hbm, d_idx_hbm, table_hbm, t_out_hbm, d_out_hbm,
                 idx_v0, idx_v1, rows_v0, rows_v1,
                 sem_i0, sem_i1, sem_g0, sem_g1, sem_o0, sem_o1):
    idx_v = (idx_v0, idx_v1)
    rows_v = (rows_v0, rows_v1)
    sem_i = (sem_i0, sem_i1)
    sem_g = (sem_g0, sem_g1)
    sem_o = (sem_o0, sem_o1)

    wid = lax.axis_index("s") * NC + lax.axis_index("c")

    def run_half(idx_hbm, out_hbm, lwid):
        base = lwid * ROWS_PER_W

        def start_idx(c, b):
            off = pl.multiple_of(base + c * CHUNK, CHUNK)
            pltpu.make_async_copy(
                idx_hbm.at[pl.ds(off, CHUNK)], idx_v[b], sem_i[b]
            ).start()

        def wait_idx(b):
            pltpu.make_async_copy(
                idx_hbm.at[pl.ds(0, CHUNK)], idx_v[b], sem_i[b]
            ).wait()

        def start_gathers(b):
            for j in range(NBLK):
                pltpu.make_async_copy(
                    table_hbm.at[idx_v[b].at[pl.ds(j * BLK, BLK)]],
                    rows_v[b].at[pl.ds(j * BLK, BLK)],
                    sem_g[b],
                ).start()

        def wait_gathers(b):
            for j in range(NBLK):
                pltpu.make_async_copy(
                    table_hbm.at[idx_v[b].at[pl.ds(j * BLK, BLK)]],
                    rows_v[b].at[pl.ds(j * BLK, BLK)],
                    sem_g[b],
                ).wait()

        def start_out(c, b):
            off = pl.multiple_of(base + c * CHUNK, CHUNK)
            pltpu.make_async_copy(
                rows_v[b], out_hbm.at[pl.ds(off, CHUNK)], sem_o[b]
            ).start()

        def wait_out(b):
            pltpu.make_async_copy(
                rows_v[b], out_hbm.at[pl.ds(0, CHUNK)], sem_o[b]
            ).wait()

        for b in range(NBUF):
            start_idx(b, b)

        def outer(i, _):
            c0 = i * NBUF
            for b in range(NBUF):
                c = c0 + b
                @pl.when(c >= NBUF)
                def _():
                    wait_out(b)

                wait_idx(b)
                start_gathers(b)
                wait_gathers(b)
                start_out(c, b)

                @pl.when(c + NBUF < NCHUNKS)
                def _():
                    start_idx(c + NBUF, b)

            return ()

        lax.fori_loop(0, NCHUNKS // NBUF, outer, ())

        for b in range(NBUF):
            wait_out(b)

    @pl.when(wid < WPA)
    def _():
        run_half(t_idx_hbm, t_out_hbm, wid)

    @pl.when(wid >= WPA)
    def _():
        run_half(d_idx_hbm, d_out_hbm, wid - WPA)


def kernel(embedding_matrix, title_ids, desc_ids):
    t_idx = jnp.transpose(title_ids).reshape(-1).astype(jnp.int32)
    d_idx = jnp.transpose(desc_ids).reshape(-1).astype(jnp.int32)
    t_out, d_out = _gather_rows(t_idx, d_idx, embedding_matrix)
    return (t_out.reshape(L, B, D), d_out.reshape(L, B, D))
